# baseline (speedup 1.0000x reference)
import functools

import jax
import jax.numpy as jnp
from jax import lax
from jax.experimental import pallas as pl
from jax.experimental.pallas import tpu as pltpu
from jax.experimental.pallas import tpu_sc as plsc

B = 16384
K = 32
NC = 2
NS = 16
NW = NC * NS
BPW = B // NW
NCHUNK = 4
CH = BPW // NCHUNK
NG = BPW // 16


@functools.cache
def _build_mf_sc():
    mesh = plsc.VectorSubcoreMesh(core_axis_name="c", subcore_axis_name="s")
    return functools.partial(
        pl.kernel,
        mesh=mesh,
        compiler_params=pltpu.CompilerParams(use_tc_tiling_on_sc=False),
        out_type=jax.ShapeDtypeStruct((B,), jnp.float32),
        scratch_types=[
            pltpu.VMEM((BPW,), jnp.int32),
            pltpu.VMEM((BPW,), jnp.int32),
            pltpu.VMEM((K * BPW,), jnp.float32),
            pltpu.VMEM((K * BPW,), jnp.float32),
            pltpu.VMEM((BPW,), jnp.float32),
            pltpu.VMEM((BPW,), jnp.float32),
            pltpu.VMEM((BPW,), jnp.float32),
            pltpu.SemaphoreType.DMA,
        ],
    )(_mf_sc)


def _mf_sc(w_hbm, u_hbm, WT_hbm, UT_hbm, Bb_hbm, Cb_hbm, out_hbm,
           idw, idu, we, ue, bb, cb, outv, sem):
    wid = lax.axis_index("s") * NC + lax.axis_index("c")
    base = wid * BPW

    pltpu.sync_copy(w_hbm.at[pl.ds(base, BPW)], idw)
    pltpu.sync_copy(u_hbm.at[pl.ds(base, BPW)], idu)

    bias_copies = []
    for j in range(NCHUNK):
        s = pl.ds(j * CH, CH)
        bias_copies.append(
            pltpu.async_copy(Bb_hbm.at[idw.at[s]], bb.at[s], sem))
        bias_copies.append(
            pltpu.async_copy(Cb_hbm.at[idu.at[s]], cb.at[s], sem))

    def k_body(k, carry):
        for j in range(NCHUNK):
            s = pl.ds(j * CH, CH)
            d = pl.ds(k * BPW + j * CH, CH)
            pltpu.async_copy(WT_hbm.at[k].at[idw.at[s]], we.at[d], sem)
            pltpu.async_copy(UT_hbm.at[k].at[idu.at[s]], ue.at[d], sem)
        return carry

    lax.fori_loop(0, K, k_body, 0)

    pltpu.make_async_copy(w_hbm.at[pl.ds(0, K * BPW)], we, sem).wait()
    pltpu.make_async_copy(w_hbm.at[pl.ds(0, K * BPW)], ue, sem).wait()
    for cp in bias_copies:
        cp.wait()

    def dot_body(t, carry):
        s = pl.ds(t * 16, 16)
        acc = bb[s] + cb[s]
        for k in range(K):
            sk = pl.ds(k * BPW + t * 16, 16)
            acc = acc + we[sk] * ue[sk]
        outv[s] = acc
        return carry

    lax.fori_loop(0, NG, dot_body, 0)

    pltpu.sync_copy(outv, out_hbm.at[pl.ds(base, BPW)])


def kernel(w, u, W_emb, U_emb, B_emb, C_emb):
    wf = w.reshape(B).astype(jnp.int32)
    uf = u.reshape(B).astype(jnp.int32)
    out = _build_mf_sc()(wf, uf, W_emb.T, U_emb.T,
                         B_emb.reshape(-1), C_emb.reshape(-1))
    return out.reshape(B, 1, 1)


# speedup vs baseline: 5.8008x; 5.8008x over previous
import functools

import jax
import jax.numpy as jnp
from jax import lax
from jax.experimental import pallas as pl
from jax.experimental.pallas import tpu as pltpu
from jax.experimental.pallas import tpu_sc as plsc

B = 16384
K = 32
NC = 2
NS = 16
NW = NC * NS
BPW = B // NW
NCHUNK = 4
CH = BPW // NCHUNK
GROUPS = BPW // 16
GPC = CH // 16

@functools.cache
def _build_mf_sc():
    mesh = plsc.VectorSubcoreMesh(core_axis_name="c", subcore_axis_name="s")
    return functools.partial(
        pl.kernel,
        mesh=mesh,
        compiler_params=pltpu.CompilerParams(use_tc_tiling_on_sc=False),
        out_type=jax.ShapeDtypeStruct((B,), jnp.float32),
        scratch_types=[
            pltpu.VMEM((NCHUNK, CH), jnp.int32),
            pltpu.VMEM((NCHUNK, CH), jnp.int32),
            pltpu.VMEM((NCHUNK, CH, K), jnp.float32),
            pltpu.VMEM((NCHUNK, CH, K), jnp.float32),
            pltpu.VMEM((NCHUNK, CH), jnp.float32),
            pltpu.VMEM((NCHUNK, CH), jnp.float32),
            pltpu.VMEM((NCHUNK, CH), jnp.float32),
            pltpu.SemaphoreType.DMA,
        ],
    )(_mf_sc)


def _mf_sc(w_hbm, u_hbm, W_hbm, U_hbm, Bb_hbm, Cb_hbm, out_hbm,
           idxw, idxu, we, ue, bb, cb, outv, sem):
    wid = lax.axis_index("s") * NC + lax.axis_index("c")
    base = wid * BPW

    for j in range(NCHUNK):
        pltpu.sync_copy(w_hbm.at[pl.ds(base + j * CH, CH)], idxw.at[j])
        pltpu.sync_copy(u_hbm.at[pl.ds(base + j * CH, CH)], idxu.at[j])

    copies = []
    for j in range(NCHUNK):
        copies.append(pltpu.async_copy(W_hbm.at[idxw.at[j]], we.at[j], sem))
        copies.append(pltpu.async_copy(U_hbm.at[idxu.at[j]], ue.at[j], sem))
        copies.append(pltpu.async_copy(Bb_hbm.at[idxw.at[j]], bb.at[j], sem))
        copies.append(pltpu.async_copy(Cb_hbm.at[idxu.at[j]], cb.at[j], sem))
    for cp in copies:
        cp.wait()

    lane = lax.iota(jnp.int32, 16)

    def hsum(x):
        for sh in (8, 4, 2, 1):
            x = x + jnp.take_along_axis(x, lane ^ sh, axis=0)
        return x

    def dot_body(g, carry):
        c = g // GPC
        r0 = (g % GPC) * 16
        acc = jnp.zeros((16,), jnp.float32)
        for j in range(16):
            r = r0 + j
            s = (we[c, r, pl.ds(0, 16)] * ue[c, r, pl.ds(0, 16)]
                 + we[c, r, pl.ds(16, 16)] * ue[c, r, pl.ds(16, 16)])
            acc = jnp.where(lane == j, hsum(s), acc)
        outv[c, pl.ds(r0, 16)] = (acc + bb[c, pl.ds(r0, 16)]
                                  + cb[c, pl.ds(r0, 16)])
        return carry

    lax.fori_loop(0, GROUPS, dot_body, 0)

    for j in range(NCHUNK):
        pltpu.sync_copy(outv.at[j], out_hbm.at[pl.ds(base + j * CH, CH)])


def kernel(w, u, W_emb, U_emb, B_emb, C_emb):
    wf = w.reshape(B).astype(jnp.int32)
    uf = u.reshape(B).astype(jnp.int32)
    out = _build_mf_sc()(wf, uf, W_emb, U_emb,
                         B_emb.reshape(-1), C_emb.reshape(-1))
    return out.reshape(B, 1, 1)
